# Initial kernel scaffold; baseline (speedup 1.0000x reference)
import functools

import jax
import jax.numpy as jnp
from jax import lax
from jax.experimental import pallas as pl
from jax.experimental.pallas import tpu as pltpu
from jax.experimental.pallas import tpu_sc as plsc

N_NODES = 10000
N_PAD = 10240
N_HEADS = 8
NC, NS = 2, 16
NW = NC * NS
EK = 128
NB = 81
T_EDGES = EK * NB
E_PAD = NW * T_EDGES

_f32 = jnp.float32
_vec_mesh = plsc.VectorSubcoreMesh(core_axis_name="c", subcore_axis_name="s")


def _dense1_body(x_ref, w_ref, ms_ref, md_ref, ha_ref, hb_ref, as_ref, ad_ref):
    h = jnp.dot(x_ref[...], w_ref[...], preferred_element_type=_f32)
    ha_ref[...] = h[:, :128]
    hb_ref[...] = h[:, 128:]
    a_s = jnp.dot(h, ms_ref[...], preferred_element_type=_f32)
    a_d = jnp.dot(h, md_ref[...], preferred_element_type=_f32)
    as_ref[...] = jnp.concatenate([a_s, a_s], axis=1)
    ad_ref[...] = jnp.concatenate([a_d, a_d], axis=1)


def _dense1(x, W1, M1s, M1d):
    R = 2000
    return pl.pallas_call(
        _dense1_body,
        grid=(N_NODES // R,),
        in_specs=[pl.BlockSpec((R, 128), lambda i: (i, 0)),
                  pl.BlockSpec((128, 256), lambda i: (0, 0)),
                  pl.BlockSpec((256, N_HEADS), lambda i: (0, 0)),
                  pl.BlockSpec((256, N_HEADS), lambda i: (0, 0))],
        out_specs=[pl.BlockSpec((R, 128), lambda i: (i, 0)),
                   pl.BlockSpec((R, 128), lambda i: (i, 0)),
                   pl.BlockSpec((R, 16), lambda i: (i, 0)),
                   pl.BlockSpec((R, 16), lambda i: (i, 0))],
        out_shape=[jax.ShapeDtypeStruct((N_NODES, 128), _f32),
                   jax.ShapeDtypeStruct((N_NODES, 128), _f32),
                   jax.ShapeDtypeStruct((N_NODES, 16), _f32),
                   jax.ShapeDtypeStruct((N_NODES, 16), _f32)],
    )(x, W1, M1s, M1d)


def _dense2_body(pa_ref, pb_ref, b1a_ref, b1b_ref, w2a_ref, w2b_ref,
                 ms_ref, md_ref, h2_ref, as_ref, ad_ref):
    va = pa_ref[0] + pa_ref[1] + b1a_ref[...]
    vb = pb_ref[0] + pb_ref[1] + b1b_ref[...]
    fa = jnp.where(va > 0, va, jnp.exp(va) - 1.0)
    fb = jnp.where(vb > 0, vb, jnp.exp(vb) - 1.0)
    h2 = (jnp.dot(fa, w2a_ref[...], preferred_element_type=_f32)
          + jnp.dot(fb, w2b_ref[...], preferred_element_type=_f32))
    h2_ref[...] = h2
    a_s = jnp.dot(h2, ms_ref[...], preferred_element_type=_f32)
    a_d = jnp.dot(h2, md_ref[...], preferred_element_type=_f32)
    as_ref[...] = jnp.concatenate([a_s, a_s], axis=1)
    ad_ref[...] = jnp.concatenate([a_d, a_d], axis=1)


def _dense2(partsA, partsB, b1a, b1b, W2a, W2b, M2s, M2d):
    R = 2048
    return pl.pallas_call(
        _dense2_body,
        grid=(N_PAD // R,),
        in_specs=[pl.BlockSpec((NC, R, 128), lambda i: (0, i, 0)),
                  pl.BlockSpec((NC, R, 128), lambda i: (0, i, 0)),
                  pl.BlockSpec((1, 128), lambda i: (0, 0)),
                  pl.BlockSpec((1, 128), lambda i: (0, 0)),
                  pl.BlockSpec((128, 128), lambda i: (0, 0)),
                  pl.BlockSpec((128, 128), lambda i: (0, 0)),
                  pl.BlockSpec((128, N_HEADS), lambda i: (0, 0)),
                  pl.BlockSpec((128, N_HEADS), lambda i: (0, 0))],
        out_specs=[pl.BlockSpec((R, 128), lambda i: (i, 0)),
                   pl.BlockSpec((R, 16), lambda i: (i, 0)),
                   pl.BlockSpec((R, 16), lambda i: (i, 0))],
        out_shape=[jax.ShapeDtypeStruct((N_PAD, 128), _f32),
                   jax.ShapeDtypeStruct((N_PAD, 16), _f32),
                   jax.ShapeDtypeStruct((N_PAD, 16), _f32)],
    )(partsA, partsB, b1a, b1b, W2a, W2b, M2s, M2d)


def _dense3_body(p_ref, b2_ref, o_ref):
    o_ref[...] = p_ref[0] + p_ref[1] + b2_ref[...]


def _dense3(parts, b2):
    R = 2048
    return pl.pallas_call(
        _dense3_body,
        grid=(N_PAD // R,),
        in_specs=[pl.BlockSpec((NC, R, 128), lambda i: (0, i, 0)),
                  pl.BlockSpec((1, 128), lambda i: (0, 0))],
        out_specs=pl.BlockSpec((R, 128), lambda i: (i, 0)),
        out_shape=jax.ShapeDtypeStruct((N_PAD, 128), _f32),
    )(parts, b2)


def _edge_w_body(asrc_hbm, adst_hbm, src_hbm, dst_hbm, zer_hbm,
                 w_hbm, dpart_hbm,
                 isrc_v, idst_v, as_v, ad_v, w_v, den_sp, gsem):
    c = lax.axis_index("c")
    s = lax.axis_index("s")
    tid = c * NS + s
    pltpu.sync_copy(zer_hbm.at[pl.ds(0, 640)], den_sp.at[pl.ds(s * 640, 640)])
    plsc.subcore_barrier()

    @pl.loop(0, NB)
    def _blocks(b):
        e0 = tid * T_EDGES + b * EK
        pltpu.sync_copy(src_hbm.at[pl.ds(e0, EK)], isrc_v)
        pltpu.sync_copy(dst_hbm.at[pl.ds(e0, EK)], idst_v)
        pltpu.async_copy(asrc_hbm.at[isrc_v], as_v, gsem).wait()
        pltpu.async_copy(adst_hbm.at[idst_v], ad_v, gsem).wait()

        @pl.loop(0, EK)
        def _edges(i):
            v = as_v[i, :] + ad_v[i, :]
            v = jnp.where(v > 0, v, 0.2 * v)
            w_v[i, :] = jnp.exp(v)

        pltpu.sync_copy(w_v, w_hbm.at[pl.ds(e0, EK)])
        pltpu.sync_copy(w_v, den_sp.at[idst_v], add=True)

    plsc.subcore_barrier()

    @pl.when(s == 0)
    def _():
        pltpu.sync_copy(den_sp, dpart_hbm.at[c])


def _edge_w(asrc, adst, src, dst, zer16):
    f = pl.kernel(
        _edge_w_body,
        out_type=(jax.ShapeDtypeStruct((E_PAD, 16), _f32),
                  jax.ShapeDtypeStruct((NC, N_PAD, 16), _f32)),
        mesh=_vec_mesh,
        scratch_types=[pltpu.VMEM((EK,), jnp.int32),
                       pltpu.VMEM((EK,), jnp.int32),
                       pltpu.VMEM((EK, 16), _f32),
                       pltpu.VMEM((EK, 16), _f32),
                       pltpu.VMEM((EK, 16), _f32),
                       pltpu.VMEM_SHARED((N_PAD, 16), _f32),
                       pltpu.SemaphoreType.DMA],
    )
    return f(asrc, adst, src, dst, zer16)


def _edge_msg_body(hoff, cph,
                   h_hbm, src_hbm, dst_hbm, w_hbm, d0_hbm, d1_hbm, zer_hbm,
                   att_hbm, opart_hbm,
                   isrc_v, idst_v, w_v, d0_v, d1_v, att_v, h_v, out_sp, gsem):
    c = lax.axis_index("c")
    s = lax.axis_index("s")
    tid = c * NS + s
    pltpu.sync_copy(zer_hbm.at[pl.ds(0, 640)], out_sp.at[pl.ds(s * 640, 640)])
    plsc.subcore_barrier()

    @pl.loop(0, NB)
    def _blocks(b):
        e0 = tid * T_EDGES + b * EK
        pltpu.sync_copy(src_hbm.at[pl.ds(e0, EK)], isrc_v)
        pltpu.sync_copy(dst_hbm.at[pl.ds(e0, EK)], idst_v)
        pltpu.sync_copy(w_hbm.at[pl.ds(e0, EK)], w_v)
        pltpu.async_copy(d0_hbm.at[idst_v], d0_v, gsem).wait()
        pltpu.async_copy(d1_hbm.at[idst_v], d1_v, gsem).wait()
        pltpu.async_copy(h_hbm.at[isrc_v], h_v, gsem).wait()

        @pl.loop(0, EK)
        def _edges(i):
            att = w_v[i, :] / (d0_v[i, :] + d1_v[i, :] + 1e-16)
            att_v[i, :] = att
            for j in range(8):
                sc = att_v[i, hoff + j // cph]
                h_v[i, pl.ds(j * 16, 16)] = h_v[i, pl.ds(j * 16, 16)] * sc

        pltpu.sync_copy(att_v, att_hbm.at[pl.ds(e0, EK)])
        pltpu.sync_copy(h_v, out_sp.at[idst_v], add=True)

    plsc.subcore_barrier()

    @pl.when(s == 0)
    def _():
        pltpu.sync_copy(out_sp, opart_hbm.at[c])


def _edge_msg(hoff, cph, h_tab, src, dst, w, d0, d1, zer128):
    f = pl.kernel(
        functools.partial(_edge_msg_body, hoff, cph),
        out_type=(jax.ShapeDtypeStruct((E_PAD, 16), _f32),
                  jax.ShapeDtypeStruct((NC, N_PAD, 128), _f32)),
        mesh=_vec_mesh,
        scratch_types=[pltpu.VMEM((EK,), jnp.int32),
                       pltpu.VMEM((EK,), jnp.int32),
                       pltpu.VMEM((EK, 16), _f32),
                       pltpu.VMEM((EK, 16), _f32),
                       pltpu.VMEM((EK, 16), _f32),
                       pltpu.VMEM((EK, 16), _f32),
                       pltpu.VMEM((EK, 128), _f32),
                       pltpu.VMEM_SHARED((N_PAD, 128), _f32),
                       pltpu.SemaphoreType.DMA],
    )
    return f(h_tab, src, dst, w, d0, d1, zer128)


def _blockdiag(att):
    h, csz = att.shape
    eye = jnp.eye(h, dtype=_f32)
    return (att[:, :, None] * eye[:, None, :]).reshape(h * csz, h)


def kernel(x, edge_index, edge_attr, W1, att_src1, att_dst1, bias1,
           W2, att_src2, att_dst2, bias2):
    n = N_NODES
    loop = jnp.arange(n, dtype=edge_index.dtype)
    ei = jnp.concatenate([edge_index, jnp.stack([loop, loop], axis=0)], axis=1)
    etot = ei.shape[1]
    pad = E_PAD - etot
    src_p = jnp.concatenate([ei[0], jnp.zeros((pad,), jnp.int32)])
    dst_p = jnp.concatenate([ei[1], jnp.full((pad,), N_NODES, jnp.int32)])

    M1s, M1d = _blockdiag(att_src1), _blockdiag(att_dst1)
    M2s, M2d = _blockdiag(att_src2), _blockdiag(att_dst2)

    zer16 = jnp.zeros((640, 16), _f32)
    zer128 = jnp.zeros((640, 128), _f32)

    hA, hB, as1, ad1 = _dense1(x, W1, M1s, M1d)
    as1p = jnp.pad(as1, ((0, N_PAD - n), (0, 0)))
    ad1p = jnp.pad(ad1, ((0, N_PAD - n), (0, 0)))

    w1, dpart1 = _edge_w(as1p, ad1p, src_p, dst_p, zer16)
    _, opartA = _edge_msg(0, 2, hA, src_p, dst_p, w1, dpart1[0], dpart1[1], zer128)
    _, opartB = _edge_msg(4, 2, hB, src_p, dst_p, w1, dpart1[0], dpart1[1], zer128)

    h2t, as2, ad2 = _dense2(opartA, opartB,
                            bias1[:128].reshape(1, 128), bias1[128:].reshape(1, 128),
                            W2[:128], W2[128:], M2s, M2d)

    w2, dpart2 = _edge_w(as2, ad2, src_p, dst_p, zer16)
    att2, opart2 = _edge_msg(0, 1, h2t, src_p, dst_p, w2, dpart2[0], dpart2[1], zer128)

    out2 = _dense3(opart2, bias2.reshape(1, 128))
    return out2[:n], att2[:etot, :8]


# speedup vs baseline: 27.5898x; 27.5898x over previous
import functools

import jax
import jax.numpy as jnp
from jax import lax
from jax.experimental import pallas as pl
from jax.experimental.pallas import tpu as pltpu
from jax.experimental.pallas import tpu_sc as plsc

N_NODES = 10000
N_PAD = 10240
N_HEADS = 8
NC, NS = 2, 16
NW = NC * NS
EK = 128
NB = 81
T_EDGES = EK * NB
E_PAD = NW * T_EDGES

_f32 = jnp.float32
_vec_mesh = plsc.VectorSubcoreMesh(core_axis_name="c", subcore_axis_name="s")
_sc_params = pltpu.CompilerParams(use_tc_tiling_on_sc=False)


def _dense1_body(x_ref, w_ref, ms_ref, md_ref, ha_ref, hb_ref, as_ref, ad_ref):
    h = jnp.dot(x_ref[...], w_ref[...], preferred_element_type=_f32)
    ha_ref[...] = h[:, :128]
    hb_ref[...] = h[:, 128:]
    a_s = jnp.dot(h, ms_ref[...], preferred_element_type=_f32)
    a_d = jnp.dot(h, md_ref[...], preferred_element_type=_f32)
    as_ref[...] = jnp.concatenate([a_s, a_s], axis=1)
    ad_ref[...] = jnp.concatenate([a_d, a_d], axis=1)


def _dense1(x, W1, M1s, M1d):
    R = 2000
    return pl.pallas_call(
        _dense1_body,
        grid=(N_NODES // R,),
        in_specs=[pl.BlockSpec((R, 128), lambda i: (i, 0)),
                  pl.BlockSpec((128, 256), lambda i: (0, 0)),
                  pl.BlockSpec((256, N_HEADS), lambda i: (0, 0)),
                  pl.BlockSpec((256, N_HEADS), lambda i: (0, 0))],
        out_specs=[pl.BlockSpec((R, 128), lambda i: (i, 0)),
                   pl.BlockSpec((R, 128), lambda i: (i, 0)),
                   pl.BlockSpec((R, 16), lambda i: (i, 0)),
                   pl.BlockSpec((R, 16), lambda i: (i, 0))],
        out_shape=[jax.ShapeDtypeStruct((N_NODES, 128), _f32),
                   jax.ShapeDtypeStruct((N_NODES, 128), _f32),
                   jax.ShapeDtypeStruct((N_NODES, 16), _f32),
                   jax.ShapeDtypeStruct((N_NODES, 16), _f32)],
    )(x, W1, M1s, M1d)


def _dense2_body(pa_ref, pb_ref, b1a_ref, b1b_ref, w2a_ref, w2b_ref,
                 ms_ref, md_ref, h2_ref, as_ref, ad_ref):
    va = pa_ref[0] + pa_ref[1] + b1a_ref[...]
    vb = pb_ref[0] + pb_ref[1] + b1b_ref[...]
    fa = jnp.where(va > 0, va, jnp.exp(va) - 1.0)
    fb = jnp.where(vb > 0, vb, jnp.exp(vb) - 1.0)
    h2 = (jnp.dot(fa, w2a_ref[...], preferred_element_type=_f32)
          + jnp.dot(fb, w2b_ref[...], preferred_element_type=_f32))
    h2_ref[...] = h2
    a_s = jnp.dot(h2, ms_ref[...], preferred_element_type=_f32)
    a_d = jnp.dot(h2, md_ref[...], preferred_element_type=_f32)
    as_ref[...] = jnp.concatenate([a_s, a_s], axis=1)
    ad_ref[...] = jnp.concatenate([a_d, a_d], axis=1)


def _dense2(partsA, partsB, b1a, b1b, W2a, W2b, M2s, M2d):
    R = 2048
    return pl.pallas_call(
        _dense2_body,
        grid=(N_PAD // R,),
        in_specs=[pl.BlockSpec((NC, R, 128), lambda i: (0, i, 0)),
                  pl.BlockSpec((NC, R, 128), lambda i: (0, i, 0)),
                  pl.BlockSpec((1, 128), lambda i: (0, 0)),
                  pl.BlockSpec((1, 128), lambda i: (0, 0)),
                  pl.BlockSpec((128, 128), lambda i: (0, 0)),
                  pl.BlockSpec((128, 128), lambda i: (0, 0)),
                  pl.BlockSpec((128, N_HEADS), lambda i: (0, 0)),
                  pl.BlockSpec((128, N_HEADS), lambda i: (0, 0))],
        out_specs=[pl.BlockSpec((R, 128), lambda i: (i, 0)),
                   pl.BlockSpec((R, 16), lambda i: (i, 0)),
                   pl.BlockSpec((R, 16), lambda i: (i, 0))],
        out_shape=[jax.ShapeDtypeStruct((N_PAD, 128), _f32),
                   jax.ShapeDtypeStruct((N_PAD, 16), _f32),
                   jax.ShapeDtypeStruct((N_PAD, 16), _f32)],
    )(partsA, partsB, b1a, b1b, W2a, W2b, M2s, M2d)


def _dense3_body(p_ref, b2_ref, o_ref):
    o_ref[...] = p_ref[0] + p_ref[1] + b2_ref[...]


def _dense3(parts, b2):
    R = 2048
    return pl.pallas_call(
        _dense3_body,
        grid=(N_PAD // R,),
        in_specs=[pl.BlockSpec((NC, R, 128), lambda i: (0, i, 0)),
                  pl.BlockSpec((1, 128), lambda i: (0, 0))],
        out_specs=pl.BlockSpec((R, 128), lambda i: (i, 0)),
        out_shape=jax.ShapeDtypeStruct((N_PAD, 128), _f32),
    )(parts, b2)


def _edge_w_body(asrc_hbm, adst_hbm, src_hbm, dst_hbm, zer_hbm,
                 w_hbm, dpart_hbm,
                 isrc_v, idst_v, as_v, ad_v, w_v, den_sp, gsem):
    c = lax.axis_index("c")
    s = lax.axis_index("s")
    tid = c * NS + s
    pltpu.sync_copy(zer_hbm.at[pl.ds(0, 640)], den_sp.at[pl.ds(s * 640, 640)])
    plsc.subcore_barrier()

    @pl.loop(0, NB)
    def _blocks(b):
        e0 = tid * T_EDGES + b * EK
        pltpu.sync_copy(src_hbm.at[pl.ds(e0, EK)], isrc_v)
        pltpu.sync_copy(dst_hbm.at[pl.ds(e0, EK)], idst_v)
        pltpu.async_copy(asrc_hbm.at[isrc_v], as_v, gsem).wait()
        pltpu.async_copy(adst_hbm.at[idst_v], ad_v, gsem).wait()

        @pl.loop(0, EK)
        def _edges(i):
            v = as_v[i, :] + ad_v[i, :]
            v = jnp.where(v > 0, v, 0.2 * v)
            w_v[i, :] = jnp.exp(v)

        pltpu.sync_copy(w_v, w_hbm.at[pl.ds(e0, EK)])
        pltpu.sync_copy(w_v, den_sp.at[idst_v], add=True)

    plsc.subcore_barrier()

    @pl.when(s == 0)
    def _():
        pltpu.sync_copy(den_sp, dpart_hbm.at[c])


def _edge_w(asrc, adst, src, dst, zer16):
    f = pl.kernel(
        _edge_w_body,
        out_type=(jax.ShapeDtypeStruct((E_PAD, 16), _f32),
                  jax.ShapeDtypeStruct((NC, N_PAD, 16), _f32)),
        mesh=_vec_mesh,
        scratch_types=[pltpu.VMEM((EK,), jnp.int32),
                       pltpu.VMEM((EK,), jnp.int32),
                       pltpu.VMEM((EK, 16), _f32),
                       pltpu.VMEM((EK, 16), _f32),
                       pltpu.VMEM((EK, 16), _f32),
                       pltpu.VMEM_SHARED((N_PAD, 16), _f32),
                       pltpu.SemaphoreType.DMA],
        compiler_params=_sc_params,
    )
    return f(asrc, adst, src, dst, zer16)


def _edge_msg_body(hoff, cph,
                   h_hbm, src_hbm, dst_hbm, w_hbm, d0_hbm, d1_hbm, zer_hbm,
                   att_hbm, opart_hbm,
                   isrc_v, idst_v, w_v, d0_v, d1_v, att_v, h_v, out_sp, gsem):
    c = lax.axis_index("c")
    s = lax.axis_index("s")
    tid = c * NS + s
    pltpu.sync_copy(zer_hbm.at[pl.ds(0, 640)], out_sp.at[pl.ds(s * 640, 640)])
    plsc.subcore_barrier()

    @pl.loop(0, NB)
    def _blocks(b):
        e0 = tid * T_EDGES + b * EK
        pltpu.sync_copy(src_hbm.at[pl.ds(e0, EK)], isrc_v)
        pltpu.sync_copy(dst_hbm.at[pl.ds(e0, EK)], idst_v)
        pltpu.sync_copy(w_hbm.at[pl.ds(e0, EK)], w_v)
        pltpu.async_copy(d0_hbm.at[idst_v], d0_v, gsem).wait()
        pltpu.async_copy(d1_hbm.at[idst_v], d1_v, gsem).wait()
        pltpu.async_copy(h_hbm.at[isrc_v], h_v, gsem).wait()

        @pl.loop(0, EK)
        def _edges(i):
            att = w_v[i, :] / (d0_v[i, :] + d1_v[i, :] + 1e-16)
            att_v[i, :] = att
            for j in range(8):
                sc = att[hoff + j // cph]
                h_v[i, pl.ds(j * 16, 16)] = h_v[i, pl.ds(j * 16, 16)] * sc

        pltpu.sync_copy(att_v, att_hbm.at[pl.ds(e0, EK)])
        pltpu.sync_copy(h_v, out_sp.at[idst_v], add=True)

    plsc.subcore_barrier()

    @pl.when(s == 0)
    def _():
        pltpu.sync_copy(out_sp, opart_hbm.at[c])


def _edge_msg(hoff, cph, h_tab, src, dst, w, d0, d1, zer128):
    f = pl.kernel(
        functools.partial(_edge_msg_body, hoff, cph),
        out_type=(jax.ShapeDtypeStruct((E_PAD, 16), _f32),
                  jax.ShapeDtypeStruct((NC, N_PAD, 128), _f32)),
        mesh=_vec_mesh,
        scratch_types=[pltpu.VMEM((EK,), jnp.int32),
                       pltpu.VMEM((EK,), jnp.int32),
                       pltpu.VMEM((EK, 16), _f32),
                       pltpu.VMEM((EK, 16), _f32),
                       pltpu.VMEM((EK, 16), _f32),
                       pltpu.VMEM((EK, 16), _f32),
                       pltpu.VMEM((EK, 128), _f32),
                       pltpu.VMEM_SHARED((N_PAD, 128), _f32),
                       pltpu.SemaphoreType.DMA],
        compiler_params=_sc_params,
    )
    return f(h_tab, src, dst, w, d0, d1, zer128)


def _blockdiag(att):
    h, csz = att.shape
    eye = jnp.eye(h, dtype=_f32)
    return (att[:, :, None] * eye[:, None, :]).reshape(h * csz, h)


def kernel(x, edge_index, edge_attr, W1, att_src1, att_dst1, bias1,
           W2, att_src2, att_dst2, bias2):
    n = N_NODES
    loop = jnp.arange(n, dtype=edge_index.dtype)
    ei = jnp.concatenate([edge_index, jnp.stack([loop, loop], axis=0)], axis=1)
    etot = ei.shape[1]
    pad = E_PAD - etot
    src_p = jnp.concatenate([ei[0], jnp.zeros((pad,), jnp.int32)])
    dst_p = jnp.concatenate([ei[1], jnp.full((pad,), N_NODES, jnp.int32)])

    M1s, M1d = _blockdiag(att_src1), _blockdiag(att_dst1)
    M2s, M2d = _blockdiag(att_src2), _blockdiag(att_dst2)

    zer16 = jnp.zeros((640, 16), _f32)
    zer128 = jnp.zeros((640, 128), _f32)

    hA, hB, as1, ad1 = _dense1(x, W1, M1s, M1d)
    as1p = jnp.pad(as1, ((0, N_PAD - n), (0, 0)))
    ad1p = jnp.pad(ad1, ((0, N_PAD - n), (0, 0)))

    w1, dpart1 = _edge_w(as1p, ad1p, src_p, dst_p, zer16)
    _, opartA = _edge_msg(0, 2, hA, src_p, dst_p, w1, dpart1[0], dpart1[1], zer128)
    _, opartB = _edge_msg(4, 2, hB, src_p, dst_p, w1, dpart1[0], dpart1[1], zer128)

    h2t, as2, ad2 = _dense2(opartA, opartB,
                            bias1[:128].reshape(1, 128), bias1[128:].reshape(1, 128),
                            W2[:128], W2[128:], M2s, M2d)

    w2, dpart2 = _edge_w(as2, ad2, src_p, dst_p, zer16)
    att2, opart2 = _edge_msg(0, 1, h2t, src_p, dst_p, w2, dpart2[0], dpart2[1], zer128)

    out2 = _dense3(opart2, bias2.reshape(1, 128))
    return out2[:n], att2[:etot, :8]


# speedup vs baseline: 46.2368x; 1.6759x over previous
import functools

import jax
import jax.numpy as jnp
from jax import lax
from jax.experimental import pallas as pl
from jax.experimental.pallas import tpu as pltpu
from jax.experimental.pallas import tpu_sc as plsc

N_NODES = 10000
N_PAD = 10240
N_HEADS = 8
NC, NS = 2, 16
NW = NC * NS
EK = 128
NB = 81
T_EDGES = EK * NB
E_PAD = NW * T_EDGES

_f32 = jnp.float32
_vec_mesh = plsc.VectorSubcoreMesh(core_axis_name="c", subcore_axis_name="s")
_sc_params = pltpu.CompilerParams(use_tc_tiling_on_sc=False)


def _dense1_body(x_ref, w_ref, ms_ref, md_ref, ha_ref, hb_ref, as_ref, ad_ref):
    h = jnp.dot(x_ref[...], w_ref[...], preferred_element_type=_f32)
    ha_ref[...] = h[:, :128]
    hb_ref[...] = h[:, 128:]
    a_s = jnp.dot(h, ms_ref[...], preferred_element_type=_f32)
    a_d = jnp.dot(h, md_ref[...], preferred_element_type=_f32)
    as_ref[...] = jnp.concatenate([a_s, a_s], axis=1)
    ad_ref[...] = jnp.concatenate([a_d, a_d], axis=1)


def _dense1(x, W1, M1s, M1d):
    R = 2000
    return pl.pallas_call(
        _dense1_body,
        grid=(N_NODES // R,),
        in_specs=[pl.BlockSpec((R, 128), lambda i: (i, 0)),
                  pl.BlockSpec((128, 256), lambda i: (0, 0)),
                  pl.BlockSpec((256, N_HEADS), lambda i: (0, 0)),
                  pl.BlockSpec((256, N_HEADS), lambda i: (0, 0))],
        out_specs=[pl.BlockSpec((R, 128), lambda i: (i, 0)),
                   pl.BlockSpec((R, 128), lambda i: (i, 0)),
                   pl.BlockSpec((R, 16), lambda i: (i, 0)),
                   pl.BlockSpec((R, 16), lambda i: (i, 0))],
        out_shape=[jax.ShapeDtypeStruct((N_NODES, 128), _f32),
                   jax.ShapeDtypeStruct((N_NODES, 128), _f32),
                   jax.ShapeDtypeStruct((N_NODES, 16), _f32),
                   jax.ShapeDtypeStruct((N_NODES, 16), _f32)],
    )(x, W1, M1s, M1d)


def _dense2_body(pa_ref, pb_ref, b1a_ref, b1b_ref, w2a_ref, w2b_ref,
                 ms_ref, md_ref, h2_ref, as_ref, ad_ref):
    va = pa_ref[0] + pa_ref[1] + b1a_ref[...]
    vb = pb_ref[0] + pb_ref[1] + b1b_ref[...]
    fa = jnp.where(va > 0, va, jnp.exp(va) - 1.0)
    fb = jnp.where(vb > 0, vb, jnp.exp(vb) - 1.0)
    h2 = (jnp.dot(fa, w2a_ref[...], preferred_element_type=_f32)
          + jnp.dot(fb, w2b_ref[...], preferred_element_type=_f32))
    h2_ref[...] = h2
    a_s = jnp.dot(h2, ms_ref[...], preferred_element_type=_f32)
    a_d = jnp.dot(h2, md_ref[...], preferred_element_type=_f32)
    as_ref[...] = jnp.concatenate([a_s, a_s], axis=1)
    ad_ref[...] = jnp.concatenate([a_d, a_d], axis=1)


def _dense2(partsA, partsB, b1a, b1b, W2a, W2b, M2s, M2d):
    R = 2048
    return pl.pallas_call(
        _dense2_body,
        grid=(N_PAD // R,),
        in_specs=[pl.BlockSpec((NC, R, 128), lambda i: (0, i, 0)),
                  pl.BlockSpec((NC, R, 128), lambda i: (0, i, 0)),
                  pl.BlockSpec((1, 128), lambda i: (0, 0)),
                  pl.BlockSpec((1, 128), lambda i: (0, 0)),
                  pl.BlockSpec((128, 128), lambda i: (0, 0)),
                  pl.BlockSpec((128, 128), lambda i: (0, 0)),
                  pl.BlockSpec((128, N_HEADS), lambda i: (0, 0)),
                  pl.BlockSpec((128, N_HEADS), lambda i: (0, 0))],
        out_specs=[pl.BlockSpec((R, 128), lambda i: (i, 0)),
                   pl.BlockSpec((R, 16), lambda i: (i, 0)),
                   pl.BlockSpec((R, 16), lambda i: (i, 0))],
        out_shape=[jax.ShapeDtypeStruct((N_PAD, 128), _f32),
                   jax.ShapeDtypeStruct((N_PAD, 16), _f32),
                   jax.ShapeDtypeStruct((N_PAD, 16), _f32)],
    )(partsA, partsB, b1a, b1b, W2a, W2b, M2s, M2d)


def _dense3_body(p_ref, b2_ref, o_ref):
    o_ref[...] = p_ref[0] + p_ref[1] + b2_ref[...]


def _dense3(parts, b2):
    R = 2048
    return pl.pallas_call(
        _dense3_body,
        grid=(N_PAD // R,),
        in_specs=[pl.BlockSpec((NC, R, 128), lambda i: (0, i, 0)),
                  pl.BlockSpec((1, 128), lambda i: (0, 0))],
        out_specs=pl.BlockSpec((R, 128), lambda i: (i, 0)),
        out_shape=jax.ShapeDtypeStruct((N_PAD, 128), _f32),
    )(parts, b2)


def _edge_w_body(asrc_hbm, adst_hbm, src_hbm, dst_hbm, zer_hbm,
                 w_hbm, dpart_hbm,
                 isrc_v, idst_v, as_v, ad_v, w_v, den_sp, gsem):
    c = lax.axis_index("c")
    s = lax.axis_index("s")
    tid = c * NS + s
    pltpu.sync_copy(zer_hbm.at[pl.ds(0, 640)], den_sp.at[pl.ds(s * 640, 640)])
    plsc.subcore_barrier()

    @pl.loop(0, NB)
    def _blocks(b):
        e0 = tid * T_EDGES + b * EK
        ia = pltpu.async_copy(src_hbm.at[pl.ds(e0, EK)], isrc_v, gsem)
        ib = pltpu.async_copy(dst_hbm.at[pl.ds(e0, EK)], idst_v, gsem)
        ia.wait()
        ib.wait()
        ga = pltpu.async_copy(asrc_hbm.at[isrc_v], as_v, gsem)
        gb = pltpu.async_copy(adst_hbm.at[idst_v], ad_v, gsem)
        ga.wait()
        gb.wait()

        @plsc.parallel_loop(0, EK, unroll=4)
        def _edges(i):
            v = as_v[i, :] + ad_v[i, :]
            v = jnp.where(v > 0, v, 0.2 * v)
            w_v[i, :] = jnp.exp(v)

        ws = pltpu.async_copy(w_v, w_hbm.at[pl.ds(e0, EK)], gsem)
        pltpu.sync_copy(w_v, den_sp.at[idst_v], add=True)
        ws.wait()

    plsc.subcore_barrier()

    @pl.when(s == 0)
    def _():
        pltpu.sync_copy(den_sp, dpart_hbm.at[c])


def _edge_w(asrc, adst, src, dst, zer16):
    f = pl.kernel(
        _edge_w_body,
        out_type=(jax.ShapeDtypeStruct((E_PAD, 16), _f32),
                  jax.ShapeDtypeStruct((NC, N_PAD, 16), _f32)),
        mesh=_vec_mesh,
        scratch_types=[pltpu.VMEM((EK,), jnp.int32),
                       pltpu.VMEM((EK,), jnp.int32),
                       pltpu.VMEM((EK, 16), _f32),
                       pltpu.VMEM((EK, 16), _f32),
                       pltpu.VMEM((EK, 16), _f32),
                       pltpu.VMEM_SHARED((N_PAD, 16), _f32),
                       pltpu.SemaphoreType.DMA],
        compiler_params=_sc_params,
    )
    return f(asrc, adst, src, dst, zer16)


def _edge_msg_body(hoff, cph, store_att,
                   h_hbm, src_hbm, dst_hbm, w_hbm, d0_hbm, d1_hbm, zer_hbm,
                   *out_and_scratch):
    if store_att:
        att_hbm, opart_hbm = out_and_scratch[:2]
        rest = out_and_scratch[2:]
    else:
        opart_hbm = out_and_scratch[0]
        rest = out_and_scratch[1:]
    (isrc_v, idst_v, w_v, d0_v, d1_v, att_v, h_v, out_sp, gsem) = rest
    _edge_msg_impl(hoff, cph, store_att,
                   h_hbm, src_hbm, dst_hbm, w_hbm, d0_hbm, d1_hbm, zer_hbm,
                   att_hbm if store_att else None, opart_hbm,
                   isrc_v, idst_v, w_v, d0_v, d1_v, att_v, h_v, out_sp, gsem)


def _edge_msg_impl(hoff, cph, store_att,
                   h_hbm, src_hbm, dst_hbm, w_hbm, d0_hbm, d1_hbm, zer_hbm,
                   att_hbm, opart_hbm,
                   isrc_v, idst_v, w_v, d0_v, d1_v, att_v, h_v, out_sp, gsem):
    c = lax.axis_index("c")
    s = lax.axis_index("s")
    tid = c * NS + s
    pltpu.sync_copy(zer_hbm.at[pl.ds(0, 640)], out_sp.at[pl.ds(s * 640, 640)])
    plsc.subcore_barrier()

    @pl.loop(0, NB)
    def _blocks(b):
        e0 = tid * T_EDGES + b * EK
        ia = pltpu.async_copy(src_hbm.at[pl.ds(e0, EK)], isrc_v, gsem)
        ib = pltpu.async_copy(dst_hbm.at[pl.ds(e0, EK)], idst_v, gsem)
        iw = pltpu.async_copy(w_hbm.at[pl.ds(e0, EK)], w_v, gsem)
        ia.wait()
        ib.wait()
        g0 = pltpu.async_copy(d0_hbm.at[idst_v], d0_v, gsem)
        g1 = pltpu.async_copy(d1_hbm.at[idst_v], d1_v, gsem)
        gh = pltpu.async_copy(h_hbm.at[isrc_v], h_v, gsem)
        iw.wait()
        g0.wait()
        g1.wait()
        gh.wait()

        @plsc.parallel_loop(0, EK, unroll=4)
        def _edges(i):
            att = w_v[i, :] / (d0_v[i, :] + d1_v[i, :] + 1e-16)
            if store_att:
                att_v[i, :] = att
            for j in range(8):
                sc = att[hoff + j // cph]
                h_v[i, pl.ds(j * 16, 16)] = h_v[i, pl.ds(j * 16, 16)] * sc

        if store_att:
            sa = pltpu.async_copy(att_v, att_hbm.at[pl.ds(e0, EK)], gsem)
        pltpu.sync_copy(h_v, out_sp.at[idst_v], add=True)
        if store_att:
            sa.wait()

    plsc.subcore_barrier()

    @pl.when(s == 0)
    def _():
        pltpu.sync_copy(out_sp, opart_hbm.at[c])


def _edge_msg(hoff, cph, store_att, h_tab, src, dst, w, d0, d1, zer128):
    if store_att:
        otype = (jax.ShapeDtypeStruct((E_PAD, 16), _f32),
                 jax.ShapeDtypeStruct((NC, N_PAD, 128), _f32))
    else:
        otype = (jax.ShapeDtypeStruct((NC, N_PAD, 128), _f32),)
    f = pl.kernel(
        functools.partial(_edge_msg_body, hoff, cph, store_att),
        out_type=otype,
        mesh=_vec_mesh,
        scratch_types=[pltpu.VMEM((EK,), jnp.int32),
                       pltpu.VMEM((EK,), jnp.int32),
                       pltpu.VMEM((EK, 16), _f32),
                       pltpu.VMEM((EK, 16), _f32),
                       pltpu.VMEM((EK, 16), _f32),
                       pltpu.VMEM((EK, 16), _f32),
                       pltpu.VMEM((EK, 128), _f32),
                       pltpu.VMEM_SHARED((N_PAD, 128), _f32),
                       pltpu.SemaphoreType.DMA],
        compiler_params=_sc_params,
    )
    return f(h_tab, src, dst, w, d0, d1, zer128)


def _blockdiag(att):
    h, csz = att.shape
    eye = jnp.eye(h, dtype=_f32)
    return (att[:, :, None] * eye[:, None, :]).reshape(h * csz, h)


def kernel(x, edge_index, edge_attr, W1, att_src1, att_dst1, bias1,
           W2, att_src2, att_dst2, bias2):
    n = N_NODES
    loop = jnp.arange(n, dtype=edge_index.dtype)
    ei = jnp.concatenate([edge_index, jnp.stack([loop, loop], axis=0)], axis=1)
    etot = ei.shape[1]
    pad = E_PAD - etot
    src_p = jnp.concatenate([ei[0], jnp.zeros((pad,), jnp.int32)])
    dst_p = jnp.concatenate([ei[1], jnp.full((pad,), N_NODES, jnp.int32)])

    M1s, M1d = _blockdiag(att_src1), _blockdiag(att_dst1)
    M2s, M2d = _blockdiag(att_src2), _blockdiag(att_dst2)

    zer16 = jnp.zeros((640, 16), _f32)
    zer128 = jnp.zeros((640, 128), _f32)

    hA, hB, as1, ad1 = _dense1(x, W1, M1s, M1d)
    as1p = jnp.pad(as1, ((0, N_PAD - n), (0, 0)))
    ad1p = jnp.pad(ad1, ((0, N_PAD - n), (0, 0)))

    w1, dpart1 = _edge_w(as1p, ad1p, src_p, dst_p, zer16)
    opartA, = _edge_msg(0, 2, False, hA, src_p, dst_p, w1, dpart1[0], dpart1[1], zer128)
    opartB, = _edge_msg(4, 2, False, hB, src_p, dst_p, w1, dpart1[0], dpart1[1], zer128)

    h2t, as2, ad2 = _dense2(opartA, opartB,
                            bias1[:128].reshape(1, 128), bias1[128:].reshape(1, 128),
                            W2[:128], W2[128:], M2s, M2d)

    w2, dpart2 = _edge_w(as2, ad2, src_p, dst_p, zer16)
    att2, opart2 = _edge_msg(0, 1, True, h2t, src_p, dst_p, w2, dpart2[0], dpart2[1], zer128)

    out2 = _dense3(opart2, bias2.reshape(1, 128))
    return out2[:n], att2[:etot, :8]
